# baseline (speedup 1.0000x reference)
import functools

import jax
import jax.numpy as jnp
from jax import lax
from jax.experimental import pallas as pl
from jax.experimental.pallas import tpu as pltpu
from jax.experimental.pallas import tpu_sc as plsc

NC = 2
NS = 16
NW = NC * NS
LANES = 128
GBUF = 4
PBUF = 2


@functools.partial(jax.jit, static_argnums=(2, 3))
def _emb_lookup(w128, idx_t, n_s, d):
    mesh = plsc.VectorSubcoreMesh(core_axis_name="c", subcore_axis_name="s")
    n_g = d // 8

    @functools.partial(
        pl.kernel,
        mesh=mesh,
        out_type=jax.ShapeDtypeStruct((n_s * n_g * NW, 8, LANES), jnp.float32),
        compiler_params=pltpu.CompilerParams(
            use_tc_tiling_on_sc=False, needs_layout_passes=False
        ),
        scratch_types=[
            pltpu.VMEM((n_s, LANES), jnp.int32),
            pltpu.VMEM((GBUF, LANES, LANES), jnp.float32),
            pltpu.VMEM((PBUF, n_g, 8, LANES), jnp.float32),
            pltpu.SemaphoreType.DMA((GBUF,)),
            pltpu.SemaphoreType.DMA((PBUF,)),
        ],
    )
    def body(table_hbm, idx_hbm, p_hbm, idx_v, g_v, p_v, gsem, osem):
        wid = lax.axis_index("s") * NC + lax.axis_index("c")
        pltpu.sync_copy(idx_hbm.at[:, pl.ds(wid * LANES, LANES)], idx_v)

        def gather_descr(s, buf):
            return pltpu.make_async_copy(
                table_hbm.at[idx_v.at[s]], g_v.at[buf], gsem.at[buf]
            )

        def out_descr(s, g, buf):
            t = (s * 8 + g) * NW + wid
            return pltpu.make_async_copy(
                p_v.at[buf, g], p_hbm.at[t], osem.at[buf]
            )

        def transpose_task(gb, pb):
            for g in range(n_g):
                for r in range(8):
                    f = 8 * g + r
                    for m in range(LANES // 16):
                        rows = lax.iota(jnp.int32, 16) + (16 * m)
                        cols = jnp.full((16,), f, jnp.int32)
                        vals = plsc.load_gather(g_v.at[gb], [rows, cols])
                        p_v[pb, g, r, pl.ds(16 * m, 16)] = vals

        for b in range(GBUF):
            gather_descr(b, b).start()

        @pl.loop(0, n_s)
        def _(s):
            gb = lax.rem(s, GBUF)
            pb = lax.rem(s, PBUF)
            gather_descr(s, gb).wait()

            @pl.when(s >= PBUF)
            def _():
                for g in range(n_g):
                    out_descr(s - PBUF, g, pb).wait()

            transpose_task(gb, pb)
            for g in range(n_g):
                out_descr(s, g, pb).start()

            @pl.when(s + GBUF < n_s)
            def _():
                gather_descr(s + GBUF, gb).start()

        @pl.loop(n_s - PBUF, n_s)
        def _(s):
            pb = lax.rem(s, PBUF)
            for g in range(n_g):
                out_descr(s, g, pb).wait()

    return body(w128, idx_t)


def kernel(input, weight):
    s0, s1 = input.shape
    v, d = weight.shape
    w128 = jnp.concatenate(
        [weight, jnp.zeros((v, LANES - d), jnp.float32)], axis=1
    )
    idx_t = input.T.astype(jnp.int32)
    p = _emb_lookup(w128, idx_t, s1, d)
    out = (
        p.reshape(s1, d // 8, s0 // LANES, 8, LANES)
        .transpose(2, 4, 0, 1, 3)
        .reshape(s0, s1, d)
    )
    return out
